# baseline (speedup 1.0000x reference)
import functools

import jax
import jax.numpy as jnp
from jax import lax
from jax.experimental import pallas as pl
from jax.experimental.pallas import tpu as pltpu
from jax.experimental.pallas import tpu_sc as plsc

B = 16384
D = 64
OUTW = 7 * D
BINS = 10000
NF = 6
NC = 2
NS = 16
NW = NC * NS
RPW = B // NW
RT = RPW // 128
NTC = OUTW // 8
NTE = D // 8
L = 16
TILE = 8 * 128
BAND = RT * TILE


def _emb_body(eid_hbm, table_hbm, out_hbm,
              idx_v, rows_v, pair0_v, pair1_v, t16_v,
              gsem, psem0, psem1):
    wid = lax.axis_index("s") * NC + lax.axis_index("c")
    base = wid * RPW
    j0 = wid * RT

    pltpu.sync_copy(eid_hbm.at[pl.ds(base, RPW)], idx_v)

    def _mod_body(i, _):
        v = idx_v[pl.ds(i * L, L)]
        idx_v[pl.ds(i * L, L)] = lax.rem(v, BINS)
        return 0

    lax.fori_loop(0, RPW // L, _mod_body, 0)

    pltpu.async_copy(table_hbm.at[idx_v], rows_v, gsem).wait()

    psems = (psem0, psem1)
    pairs = (pair0_v, pair1_v)
    lanes16 = lax.iota(jnp.int32, L)
    colpat = [(lanes16 + k) % L for k in range(L)]
    qpat = [18 * ((c - lanes16) % L) + lanes16 for c in range(L)]

    def _pair_wait(buf, sem):
        pltpu.make_async_copy(
            pairs[buf], out_hbm.at[pl.ds(0, 2 * BAND)], sem).wait()

    def _emb_pair(p, buf, sem):
        pair = pairs[buf]
        cidx = [cp + p * L for cp in colpat]

        def _rows(q, _):
            rr = q * L
            jj = rr // 128
            sbase = jj * TILE + (rr - jj * 128)
            ridx = lanes16 + rr
            for k in range(L):
                t16_v[pl.ds(k * 18, L)] = plsc.load_gather(
                    rows_v, [ridx, cidx[k]])
            for c in range(L):
                v = plsc.load_gather(t16_v, [qpat[c]])
                pair[pl.ds((c // 8) * BAND + (c % 8) * 128 + sbase, L)] = v
            return 0

        lax.fori_loop(0, RPW // L, _rows, 0)
        pltpu.async_copy(
            pair.at[pl.ds(0, BAND)],
            out_hbm.at[pl.ds((2 * p * 128 + j0) * TILE, BAND)], sem)
        pltpu.async_copy(
            pair.at[pl.ds(BAND, BAND)],
            out_hbm.at[pl.ds(((2 * p + 1) * 128 + j0) * TILE, BAND)], sem)

    for p in range(4):
        if p >= 2:
            _pair_wait(p % 2, psems[p % 2])
        _emb_pair(p, p % 2, psems[p % 2])

    _pair_wait(0, psems[0])
    _pair_wait(1, psems[1])


@functools.partial(
    pl.kernel,
    mesh=plsc.VectorSubcoreMesh(core_axis_name="c", subcore_axis_name="s"),
    out_type=jax.ShapeDtypeStruct((B * D,), jnp.float32),
    compiler_params=pltpu.CompilerParams(use_tc_tiling_on_sc=False,
                                         needs_layout_passes=False),
    scratch_types=[
        pltpu.VMEM((RPW,), jnp.int32),
        pltpu.VMEM((RPW, D), jnp.float32),
        pltpu.VMEM((2 * BAND,), jnp.float32),
        pltpu.VMEM((2 * BAND,), jnp.float32),
        pltpu.VMEM((16 * 18,), jnp.float32),
        pltpu.SemaphoreType.DMA,
        pltpu.SemaphoreType.DMA,
        pltpu.SemaphoreType.DMA,
    ],
)
def _emb_kernel(eid, table, out, idx_v, rows_v, pair0_v, pair1_v, t16_v,
                gsem, psem0, psem1):
    _emb_body(eid, table, out, idx_v, rows_v, pair0_v, pair1_v, t16_v,
              gsem, psem0, psem1)


def _bands_tc_body(emb_ref, feat_ref, w_ref, b_ref, out_ref):
    i = pl.program_id(0)

    @pl.when(i < NTE)
    def _copy():
        out_ref[0] = emb_ref[0]

    @pl.when(i >= NTE)
    def _dense():
        f = feat_ref[0]
        ft = jnp.concatenate([f] * 8, axis=1)
        out_ref[0] = ft * w_ref[0] + b_ref[0]


_bands_tc = pl.pallas_call(
    _bands_tc_body,
    grid=(NTC,),
    in_specs=[
        pl.BlockSpec((1, 128, 1024), lambda i: (jnp.minimum(i, NTE - 1), 0, 0)),
        pl.BlockSpec((1, 128, 128),
                     lambda i: (jnp.maximum(i - NTE, 0) // NTE, 0, 0)),
        pl.BlockSpec((1, 1, 1024), lambda i: (jnp.maximum(i - NTE, 0), 0, 0)),
        pl.BlockSpec((1, 1, 1024), lambda i: (jnp.maximum(i - NTE, 0), 0, 0)),
    ],
    out_specs=pl.BlockSpec((1, 128, 1024), lambda i: (i, 0, 0)),
    out_shape=jax.ShapeDtypeStruct((NTC, 128, 1024), jnp.float32),
)


def kernel(engagement_id, table,
           feat_type, W_type, b_type,
           feat_duration, W_duration, b_duration,
           feat_difficulty, W_difficulty, b_difficulty,
           feat_prerequisites, W_prerequisites, b_prerequisites,
           feat_popularity, W_popularity, b_popularity,
           feat_success_rate, W_success_rate, b_success_rate):
    feats3 = jnp.stack([feat_type, feat_duration, feat_difficulty,
                        feat_prerequisites, feat_popularity,
                        feat_success_rate]).reshape(NF, 128, 128)
    w = jnp.concatenate([W_type[0], W_duration[0], W_difficulty[0],
                         W_prerequisites[0], W_popularity[0],
                         W_success_rate[0]])
    bb = jnp.concatenate([b_type, b_duration, b_difficulty,
                          b_prerequisites, b_popularity, b_success_rate])
    w_bands = jnp.broadcast_to(
        w.reshape(NTC - NTE, 8)[:, :, None], (NTC - NTE, 8, 128)
    ).reshape(NTC - NTE, 1, 1024)
    b_bands = jnp.broadcast_to(
        bb.reshape(NTC - NTE, 8)[:, :, None], (NTC - NTE, 8, 128)
    ).reshape(NTC - NTE, 1, 1024)

    emb3 = _emb_kernel(engagement_id, table).reshape(NTE, 128, 1024)
    out3 = _bands_tc(emb3, feats3, w_bands, b_bands)
    return (out3.reshape(NTC, B // 128, 8, 128)
            .transpose(1, 3, 0, 2)
            .reshape(B, OUTW))


# speedup vs baseline: 2.0482x; 2.0482x over previous
import functools

import jax
import jax.numpy as jnp
from jax import lax
from jax.experimental import pallas as pl
from jax.experimental.pallas import tpu as pltpu
from jax.experimental.pallas import tpu_sc as plsc

B = 16384
D = 64
OUTW = 7 * D
BINS = 10000
NF = 6
NC = 2
NS = 16
NW = NC * NS
RPW = B // NW
RT = RPW // 128
NTC = OUTW // 8
L = 16
TILE = 8 * 128
BAND = RT * TILE


def _tower_body(eid_hbm, table_hbm, feat_refs, w_refs, b_refs, out_hbm,
                idx_v, rows_v, feats_v, w_v, b_v, band_v, pair0_v, pair1_v,
                t16_v, gsem, bsem0, bsem1, psem0, psem1):
    wid = lax.axis_index("s") * NC + lax.axis_index("c")
    base = wid * RPW
    j0 = wid * RT

    pltpu.sync_copy(eid_hbm.at[pl.ds(base, RPW)], idx_v)

    def _mod_body(i, _):
        v = idx_v[pl.ds(i * L, L)]
        idx_v[pl.ds(i * L, L)] = lax.rem(v, BINS)
        return 0

    lax.fori_loop(0, RPW // L, _mod_body, 0)

    gather = pltpu.async_copy(table_hbm.at[idx_v], rows_v, gsem)

    for f in range(NF):
        pltpu.sync_copy(w_refs[f].at[0], w_v.at[pl.ds(f * D, D)])
        pltpu.sync_copy(b_refs[f], b_v.at[pl.ds(f * D, D)])
        pltpu.sync_copy(feat_refs[f].at[pl.ds(base, RPW)],
                        feats_v.at[pl.ds(f * RPW, RPW)])

    bsems = (bsem0, bsem1)

    def _band_wait(buf, sem):
        pltpu.make_async_copy(
            band_v.at[buf], out_hbm.at[pl.ds(0, BAND)], sem).wait()

    def _dense_band(i, half, buf, sem):
        band = band_v.at[buf]
        k = (i - 8) * 8
        f = k // D
        c_in_f = k - f * D
        ch16 = c_in_f - 8 * half
        wch = w_v[pl.ds(f * D + ch16, L)]
        bch = b_v[pl.ds(f * D + ch16, L)]
        wb = []
        bb = []
        for c in range(8):
            lane = jnp.full((L, 1), half * 8 + c, jnp.int32)
            dn = lax.GatherDimensionNumbers(
                offset_dims=(), collapsed_slice_dims=(0,),
                start_index_map=(0,))
            wb.append(lax.gather(wch, lane, dn, slice_sizes=(1,),
                                 mode=lax.GatherScatterMode.PROMISE_IN_BOUNDS))
            bb.append(lax.gather(bch, lane, dn, slice_sizes=(1,),
                                 mode=lax.GatherScatterMode.PROMISE_IN_BOUNDS))
        for jj in range(RT):
            fch = [feats_v[pl.ds(f * RPW + jj * 128 + h * L, L)]
                   for h in range(8)]
            for c in range(8):
                for h in range(8):
                    band[pl.ds(jj * TILE + c * 128 + h * L, L)] = (
                        fch[h] * wb[c] + bb[c])
        pltpu.async_copy(
            band, out_hbm.at[pl.ds((i * 128 + j0) * TILE, BAND)], sem)

    def _dense_pair(p, _):
        i = 8 + 2 * p
        pl.when(p > 0)(lambda: _band_wait(0, bsems[0]))
        _dense_band(i, 0, 0, bsems[0])
        pl.when(p > 0)(lambda: _band_wait(1, bsems[1]))
        _dense_band(i + 1, 1, 1, bsems[1])
        return 0

    lax.fori_loop(0, (NTC - 8) // 2, _dense_pair, 0)

    gather.wait()
    psems = (psem0, psem1)
    pairs = (pair0_v, pair1_v)
    lanes16 = lax.iota(jnp.int32, L)
    colpat = [(lanes16 + k) % L for k in range(L)]
    qpat = [18 * ((c - lanes16) % L) + lanes16 for c in range(L)]

    def _pair_wait(buf, sem):
        pltpu.make_async_copy(
            pairs[buf], out_hbm.at[pl.ds(0, 2 * BAND)], sem).wait()

    def _emb_pair(p, buf, sem):
        pair = pairs[buf]
        cidx = [cp + p * L for cp in colpat]

        def _rows(q, _):
            rr = q * L
            jj = rr // 128
            sbase = jj * TILE + (rr - jj * 128)
            ridx = lanes16 + rr
            for k in range(L):
                t16_v[pl.ds(k * 18, L)] = plsc.load_gather(
                    rows_v, [ridx, cidx[k]])
            for c in range(L):
                v = plsc.load_gather(t16_v, [qpat[c]])
                pair[pl.ds((c // 8) * BAND + (c % 8) * 128 + sbase, L)] = v
            return 0

        lax.fori_loop(0, RPW // L, _rows, 0)
        pltpu.async_copy(
            pair.at[pl.ds(0, BAND)],
            out_hbm.at[pl.ds((2 * p * 128 + j0) * TILE, BAND)], sem)
        pltpu.async_copy(
            pair.at[pl.ds(BAND, BAND)],
            out_hbm.at[pl.ds(((2 * p + 1) * 128 + j0) * TILE, BAND)], sem)

    for p in range(4):
        if p >= 2:
            _pair_wait(p % 2, psems[p % 2])
        _emb_pair(p, p % 2, psems[p % 2])

    _band_wait(0, bsems[0])
    _band_wait(1, bsems[1])
    _pair_wait(0, psems[0])
    _pair_wait(1, psems[1])


@functools.partial(
    pl.kernel,
    mesh=plsc.VectorSubcoreMesh(core_axis_name="c", subcore_axis_name="s"),
    out_type=jax.ShapeDtypeStruct((B * OUTW,), jnp.float32),
    compiler_params=pltpu.CompilerParams(use_tc_tiling_on_sc=False,
                                         needs_layout_passes=False),
    scratch_types=[
        pltpu.VMEM((RPW,), jnp.int32),
        pltpu.VMEM((RPW, D), jnp.float32),
        pltpu.VMEM((NF * RPW,), jnp.float32),
        pltpu.VMEM((NF * D,), jnp.float32),
        pltpu.VMEM((NF * D,), jnp.float32),
        pltpu.VMEM((2, BAND), jnp.float32),
        pltpu.VMEM((2 * BAND,), jnp.float32),
        pltpu.VMEM((2 * BAND,), jnp.float32),
        pltpu.VMEM((16 * 18,), jnp.float32),
        pltpu.SemaphoreType.DMA,
        pltpu.SemaphoreType.DMA,
        pltpu.SemaphoreType.DMA,
        pltpu.SemaphoreType.DMA,
        pltpu.SemaphoreType.DMA,
    ],
)
def _tower_kernel(eid, table,
                  f0, w0, b0, f1, w1, b1, f2, w2, b2,
                  f3, w3, b3, f4, w4, b4, f5, w5, b5,
                  out,
                  idx_v, rows_v, feats_v, w_v, b_v, band_v, pair0_v, pair1_v,
                  t16_v, gsem, bsem0, bsem1, psem0, psem1):
    _tower_body(eid, table,
                (f0, f1, f2, f3, f4, f5),
                (w0, w1, w2, w3, w4, w5),
                (b0, b1, b2, b3, b4, b5),
                out,
                idx_v, rows_v, feats_v, w_v, b_v, band_v, pair0_v, pair1_v,
                t16_v, gsem, bsem0, bsem1, psem0, psem1)


def kernel(engagement_id, table,
           feat_type, W_type, b_type,
           feat_duration, W_duration, b_duration,
           feat_difficulty, W_difficulty, b_difficulty,
           feat_prerequisites, W_prerequisites, b_prerequisites,
           feat_popularity, W_popularity, b_popularity,
           feat_success_rate, W_success_rate, b_success_rate):
    flat = _tower_kernel(
        engagement_id, table,
        feat_type, W_type, b_type,
        feat_duration, W_duration, b_duration,
        feat_difficulty, W_difficulty, b_difficulty,
        feat_prerequisites, W_prerequisites, b_prerequisites,
        feat_popularity, W_popularity, b_popularity,
        feat_success_rate, W_success_rate, b_success_rate)
    return (flat.reshape(NTC, B // 128, 8, 128)
            .transpose(1, 3, 0, 2)
            .reshape(B, OUTW))
